# Initial kernel scaffold; baseline (speedup 1.0000x reference)
import jax
import jax.numpy as jnp
from jax.experimental import pallas as pl


def kernel(x, edge_index, edge_type, W1, root1, b1, W2, root2, b2):
    raise NotImplementedError("write your pallas kernel here")


# speedup vs baseline: 11.5221x; 11.5221x over previous
import functools

import jax
import jax.numpy as jnp
from jax import lax
from jax.experimental import pallas as pl
from jax.experimental.pallas import tpu as pltpu
from jax.experimental.pallas import tpu_sc as plsc

N = 10000
D = 128
E = 320000
NPAD = 10240
EPAD = 327680
CHUNK = 128
CPW = 80
NTILE = 16
RPT = NPAD // NTILE
BLK = 512
LANES = 16
ZR = 16


def _sc_segsum(compute_deg):
    mesh = plsc.VectorSubcoreMesh(core_axis_name="c", subcore_axis_name="s")
    out_type = [jax.ShapeDtypeStruct((2, NPAD, D), jnp.float32)]
    scratch = [
        pltpu.VMEM((CPW, CHUNK), jnp.int32),
        pltpu.VMEM((CPW, CHUNK), jnp.int32),
        pltpu.VMEM((CHUNK, D), jnp.float32),
        pltpu.VMEM((ZR, D), jnp.float32),
        pltpu.VMEM_SHARED((NPAD, D), jnp.float32),
    ]
    if compute_deg:
        out_type.append(jax.ShapeDtypeStruct((2, NPAD), jnp.float32))
        scratch.append(pltpu.VMEM((CHUNK,), jnp.float32))
        scratch.append(pltpu.VMEM_SHARED((NPAD,), jnp.float32))
    scratch.append(pltpu.SemaphoreType.DMA)

    def body(x_hbm, src_hbm, dst_hbm, *rest):
        if compute_deg:
            (sums_out, deg_out, sidx, didx, rows, zrow, acc, ones, degacc,
             sem) = rest
        else:
            sums_out, sidx, didx, rows, zrow, acc, sem = rest
            deg_out = ones = degacc = None
        cid = lax.axis_index("c")
        tid = lax.axis_index("s")

        def zfill(i, carry):
            for j in range(D // LANES):
                zrow[i, pl.ds(j * LANES, LANES)] = jnp.zeros((LANES,),
                                                             jnp.float32)
            return carry
        lax.fori_loop(0, ZR, zfill, 0)
        if compute_deg:
            for j in range(CHUNK // LANES):
                ones[pl.ds(j * LANES, LANES)] = jnp.full((LANES,), 1.0,
                                                         jnp.float32)

        row0 = tid * RPT

        def zacc(k, carry):
            pltpu.sync_copy(zrow, acc.at[pl.ds(row0 + k * ZR, ZR), :])
            return carry
        lax.fori_loop(0, RPT // ZR, zacc, 0)
        if compute_deg:
            for k in range(RPT // D):
                pltpu.sync_copy(zrow.at[0],
                                degacc.at[pl.ds(row0 + k * D, D)])
        plsc.subcore_barrier()

        w = cid * NTILE + tid
        pltpu.sync_copy(src_hbm.at[pl.ds(w * CPW, CPW), :], sidx)
        pltpu.sync_copy(dst_hbm.at[pl.ds(w * CPW, CPW), :], didx)

        def step(i, carry):
            pltpu.async_copy(x_hbm.at[sidx.at[i]], rows, sem).wait()
            pltpu.sync_copy(rows, acc.at[didx.at[i]], add=True)
            if compute_deg:
                pltpu.sync_copy(ones, degacc.at[didx.at[i]], add=True)
            return carry
        lax.fori_loop(0, CPW, step, 0)
        plsc.subcore_barrier()

        for k in range(RPT // CHUNK):
            sl = pl.ds(row0 + k * CHUNK, CHUNK)
            pltpu.sync_copy(acc.at[sl, :], sums_out.at[cid, sl, :])
        if compute_deg:
            pltpu.sync_copy(degacc.at[pl.ds(row0, RPT)],
                            deg_out.at[cid, pl.ds(row0, RPT)])

    return pl.kernel(body, out_type=tuple(out_type), mesh=mesh,
                     scratch_types=tuple(scratch))


def _tc_body(sums_ref, deg0_ref, deg1_ref, x_ref, w_ref, r_ref, b_ref, o_ref,
             *, relu):
    cnt = deg0_ref[...] + deg1_ref[...]
    inv = (1.0 / jnp.maximum(cnt, 1.0)).reshape(1, BLK)
    ii = lax.broadcasted_iota(jnp.int32, (BLK, BLK), 0)
    jj = lax.broadcasted_iota(jnp.int32, (BLK, BLK), 1)
    dm = jnp.where(ii == jj, jnp.broadcast_to(inv, (BLK, BLK)), 0.0)
    invcol = jnp.sum(dm, axis=1, keepdims=True)
    s = sums_ref[0] + sums_ref[1]
    mean = s * invcol
    acc = jnp.dot(mean, w_ref[...], preferred_element_type=jnp.float32)
    acc = acc + jnp.dot(x_ref[...], r_ref[...],
                        preferred_element_type=jnp.float32)
    acc = acc + b_ref[...]
    if relu:
        acc = jnp.maximum(acc, 0.0)
    o_ref[...] = acc


def _tc_layer(relu):
    grid = NPAD // BLK
    return pl.pallas_call(
        functools.partial(_tc_body, relu=relu),
        grid=(grid,),
        in_specs=[
            pl.BlockSpec((2, BLK, D), lambda i: (0, i, 0)),
            pl.BlockSpec((BLK,), lambda i: (i,)),
            pl.BlockSpec((BLK,), lambda i: (i,)),
            pl.BlockSpec((BLK, D), lambda i: (i, 0)),
            pl.BlockSpec((D, D), lambda i: (0, 0)),
            pl.BlockSpec((D, D), lambda i: (0, 0)),
            pl.BlockSpec((1, D), lambda i: (0, 0)),
        ],
        out_specs=pl.BlockSpec((BLK, D), lambda i: (i, 0)),
        out_shape=jax.ShapeDtypeStruct((NPAD, D), jnp.float32),
    )


def kernel(x, edge_index, edge_type, W1, root1, b1, W2, root2, b2):
    del edge_type
    src = edge_index[0]
    dst = edge_index[1]
    pad = EPAD - E
    srcp = jnp.concatenate([src, jnp.zeros((pad,), jnp.int32)])
    dstp = jnp.concatenate([dst, jnp.full((pad,), N, jnp.int32)])
    srcp = srcp.reshape(EPAD // CHUNK, CHUNK)
    dstp = dstp.reshape(EPAD // CHUNK, CHUNK)
    xpad = jnp.concatenate([x, jnp.zeros((NPAD - N, D), x.dtype)])

    res = _sc_segsum(True)(xpad, srcp, dstp)
    sums1, deg = res
    deg0 = deg[0]
    deg1 = deg[1]
    h = _tc_layer(True)(sums1, deg0, deg1, xpad, W1[0], root1,
                        b1.reshape(1, D))
    res2 = _sc_segsum(False)(h, srcp, dstp)
    sums2 = res2[0] if isinstance(res2, (tuple, list)) else res2
    out = _tc_layer(False)(sums2, deg0, deg1, h, W2[0], root2,
                           b2.reshape(1, D))
    return out[:N]


# speedup vs baseline: 12.4839x; 1.0835x over previous
import functools

import jax
import jax.numpy as jnp
from jax import lax
from jax.experimental import pallas as pl
from jax.experimental.pallas import tpu as pltpu
from jax.experimental.pallas import tpu_sc as plsc

N = 10000
D = 128
E = 320000
NPAD = 10240
EPAD = 327680
CHUNK = 64
CPW = 160
GRP = 8
NGRP = CPW // GRP
NTILE = 16
RPT = NPAD // NTILE
BLK = 512
LANES = 16
ZR = 16


def _sc_segsum(compute_deg):
    mesh = plsc.VectorSubcoreMesh(core_axis_name="c", subcore_axis_name="s")
    out_type = [jax.ShapeDtypeStruct((2, NPAD, D), jnp.float32)]
    scratch = [
        pltpu.VMEM((GRP, CHUNK), jnp.int32),
        pltpu.VMEM((GRP, CHUNK), jnp.int32),
        pltpu.VMEM((CHUNK, D), jnp.float32),
        pltpu.VMEM((CHUNK, D), jnp.float32),
        pltpu.VMEM((ZR, D), jnp.float32),
        pltpu.VMEM_SHARED((NPAD, D), jnp.float32),
    ]
    if compute_deg:
        out_type.append(jax.ShapeDtypeStruct((2, NPAD), jnp.float32))
        scratch.append(pltpu.VMEM((CHUNK,), jnp.float32))
        scratch.append(pltpu.VMEM_SHARED((NPAD,), jnp.float32))
    scratch.append(pltpu.SemaphoreType.DMA)
    scratch.append(pltpu.SemaphoreType.DMA)

    def body(x_hbm, src_hbm, dst_hbm, *rest):
        if compute_deg:
            (sums_out, deg_out, sidx, didx, rows0, rows1, zrow, acc, ones,
             degacc, sem0, sem1) = rest
        else:
            sums_out, sidx, didx, rows0, rows1, zrow, acc, sem0, sem1 = rest
            deg_out = ones = degacc = None
        cid = lax.axis_index("c")
        tid = lax.axis_index("s")

        def zfill(i, carry):
            for j in range(D // LANES):
                zrow[i, pl.ds(j * LANES, LANES)] = jnp.zeros((LANES,),
                                                             jnp.float32)
            return carry
        lax.fori_loop(0, ZR, zfill, 0)
        if compute_deg:
            for j in range(CHUNK // LANES):
                ones[pl.ds(j * LANES, LANES)] = jnp.full((LANES,), 1.0,
                                                         jnp.float32)

        row0 = tid * RPT

        def zacc(k, carry):
            pltpu.sync_copy(zrow, acc.at[pl.ds(row0 + k * ZR, ZR), :])
            return carry
        lax.fori_loop(0, RPT // ZR, zacc, 0)
        if compute_deg:
            for k in range(RPT // D):
                pltpu.sync_copy(zrow.at[0],
                                degacc.at[pl.ds(row0 + k * D, D)])
        plsc.subcore_barrier()

        w = cid * NTILE + tid

        def consume(rows, j):
            pltpu.sync_copy(rows, acc.at[didx.at[j]], add=True)
            if compute_deg:
                pltpu.sync_copy(ones, degacc.at[didx.at[j]], add=True)

        def group(g, carry):
            grow = w * CPW + g * GRP
            pltpu.sync_copy(src_hbm.at[pl.ds(grow, GRP), :], sidx)
            pltpu.sync_copy(dst_hbm.at[pl.ds(grow, GRP), :], didx)
            bufs = (rows0, rows1)
            sems = (sem0, sem1)
            pend = [None, None]
            for j in range(GRP):
                p = j % 2
                if pend[p] is not None:
                    pend[p][0].wait()
                    consume(bufs[p], pend[p][1])
                pend[p] = (pltpu.async_copy(x_hbm.at[sidx.at[j]], bufs[p],
                                            sems[p]), j)
            for p in (0, 1):
                pend[p][0].wait()
                consume(bufs[p], pend[p][1])
            return carry
        lax.fori_loop(0, NGRP, group, 0)
        plsc.subcore_barrier()

        for k in range(RPT // CHUNK):
            sl = pl.ds(row0 + k * CHUNK, CHUNK)
            pltpu.sync_copy(acc.at[sl, :], sums_out.at[cid, sl, :])
        if compute_deg:
            pltpu.sync_copy(degacc.at[pl.ds(row0, RPT)],
                            deg_out.at[cid, pl.ds(row0, RPT)])

    return pl.kernel(body, out_type=tuple(out_type), mesh=mesh,
                     scratch_types=tuple(scratch))


def _tc_body(sums_ref, deg0_ref, deg1_ref, x_ref, w_ref, r_ref, b_ref, o_ref,
             *, relu):
    cnt = deg0_ref[...] + deg1_ref[...]
    inv = (1.0 / jnp.maximum(cnt, 1.0)).reshape(1, BLK)
    ii = lax.broadcasted_iota(jnp.int32, (BLK, BLK), 0)
    jj = lax.broadcasted_iota(jnp.int32, (BLK, BLK), 1)
    dm = jnp.where(ii == jj, jnp.broadcast_to(inv, (BLK, BLK)), 0.0)
    invcol = jnp.sum(dm, axis=1, keepdims=True)
    s = sums_ref[0] + sums_ref[1]
    mean = s * invcol
    acc = jnp.dot(mean, w_ref[...], preferred_element_type=jnp.float32)
    acc = acc + jnp.dot(x_ref[...], r_ref[...],
                        preferred_element_type=jnp.float32)
    acc = acc + b_ref[...]
    if relu:
        acc = jnp.maximum(acc, 0.0)
    o_ref[...] = acc


def _tc_layer(relu):
    grid = NPAD // BLK
    return pl.pallas_call(
        functools.partial(_tc_body, relu=relu),
        grid=(grid,),
        in_specs=[
            pl.BlockSpec((2, BLK, D), lambda i: (0, i, 0)),
            pl.BlockSpec((BLK,), lambda i: (i,)),
            pl.BlockSpec((BLK,), lambda i: (i,)),
            pl.BlockSpec((BLK, D), lambda i: (i, 0)),
            pl.BlockSpec((D, D), lambda i: (0, 0)),
            pl.BlockSpec((D, D), lambda i: (0, 0)),
            pl.BlockSpec((1, D), lambda i: (0, 0)),
        ],
        out_specs=pl.BlockSpec((BLK, D), lambda i: (i, 0)),
        out_shape=jax.ShapeDtypeStruct((NPAD, D), jnp.float32),
    )


def kernel(x, edge_index, edge_type, W1, root1, b1, W2, root2, b2):
    del edge_type
    src = edge_index[0]
    dst = edge_index[1]
    pad = EPAD - E
    srcp = jnp.concatenate([src, jnp.zeros((pad,), jnp.int32)])
    dstp = jnp.concatenate([dst, jnp.full((pad,), N, jnp.int32)])
    srcp = srcp.reshape(EPAD // CHUNK, CHUNK)
    dstp = dstp.reshape(EPAD // CHUNK, CHUNK)
    xpad = jnp.concatenate([x, jnp.zeros((NPAD - N, D), x.dtype)])

    res = _sc_segsum(True)(xpad, srcp, dstp)
    sums1, deg = res
    deg0 = deg[0]
    deg1 = deg[1]
    h = _tc_layer(True)(sums1, deg0, deg1, xpad, W1[0], root1,
                        b1.reshape(1, D))
    res2 = _sc_segsum(False)(h, srcp, dstp)
    sums2 = res2[0] if isinstance(res2, (tuple, list)) else res2
    out = _tc_layer(False)(sums2, deg0, deg1, h, W2[0], root2,
                           b2.reshape(1, D))
    return out[:N]


# speedup vs baseline: 13.2739x; 1.0633x over previous
import functools

import jax
import jax.numpy as jnp
from jax import lax
from jax.experimental import pallas as pl
from jax.experimental.pallas import tpu as pltpu
from jax.experimental.pallas import tpu_sc as plsc

N = 10000
D = 128
E = 320000
NPAD = 10240
EPAD = 327680
CHUNK = 64
CPW = 160
GRP = 16
NGRP = CPW // GRP
NBUF = 4
DEPTH = 2
NTILE = 16
RPT = NPAD // NTILE
BLK = 512
LANES = 16
ZR = 16


def _sc_segsum(compute_deg):
    mesh = plsc.VectorSubcoreMesh(core_axis_name="c", subcore_axis_name="s")
    out_type = [jax.ShapeDtypeStruct((2, NPAD, D), jnp.float32)]
    scratch = [
        pltpu.VMEM((GRP, CHUNK), jnp.int32),
        pltpu.VMEM((GRP, CHUNK), jnp.int32),
        pltpu.VMEM((ZR, D), jnp.float32),
        pltpu.VMEM_SHARED((NPAD, D), jnp.float32),
    ]
    scratch += [pltpu.VMEM((CHUNK, D), jnp.float32)] * NBUF
    if compute_deg:
        out_type.append(jax.ShapeDtypeStruct((2, NPAD), jnp.float32))
        scratch.append(pltpu.VMEM((CHUNK,), jnp.float32))
        scratch.append(pltpu.VMEM_SHARED((NPAD,), jnp.float32))
    scratch += [pltpu.SemaphoreType.DMA] * (2 * NBUF + 1)

    def body(x_hbm, src_hbm, dst_hbm, *rest):
        if compute_deg:
            (sums_out, deg_out, sidx, didx, zrow, acc, *tl) = rest
            bufs = tl[:NBUF]
            ones, degacc = tl[NBUF], tl[NBUF + 1]
            sems = tl[NBUF + 2:]
        else:
            (sums_out, sidx, didx, zrow, acc, *tl) = rest
            bufs = tl[:NBUF]
            deg_out = ones = degacc = None
            sems = tl[NBUF:]
        gsems = sems[:NBUF]
        ssems = sems[NBUF:2 * NBUF]
        degsem = sems[2 * NBUF]
        cid = lax.axis_index("c")
        tid = lax.axis_index("s")

        def zfill(i, carry):
            for j in range(D // LANES):
                zrow[i, pl.ds(j * LANES, LANES)] = jnp.zeros((LANES,),
                                                             jnp.float32)
            return carry
        lax.fori_loop(0, ZR, zfill, 0)
        if compute_deg:
            for j in range(CHUNK // LANES):
                ones[pl.ds(j * LANES, LANES)] = jnp.full((LANES,), 1.0,
                                                         jnp.float32)

        row0 = tid * RPT

        def zacc(k, carry):
            pltpu.sync_copy(zrow, acc.at[pl.ds(row0 + k * ZR, ZR), :])
            return carry
        lax.fori_loop(0, RPT // ZR, zacc, 0)
        if compute_deg:
            for k in range(RPT // D):
                pltpu.sync_copy(zrow.at[0],
                                degacc.at[pl.ds(row0 + k * D, D)])
        plsc.subcore_barrier()

        w = cid * NTILE + tid

        def group(g, carry):
            grow = w * CPW + g * GRP
            pltpu.sync_copy(src_hbm.at[pl.ds(grow, GRP), :], sidx)
            pltpu.sync_copy(dst_hbm.at[pl.ds(grow, GRP), :], didx)
            gd = [None] * GRP
            sd = [None] * GRP
            degd = []

            def fire_scatter(j):
                p = j % NBUF
                gd[j].wait()
                sd[j] = pltpu.async_copy(bufs[p], acc.at[didx.at[j]],
                                         ssems[p], add=True)
                if compute_deg:
                    degd.append(
                        pltpu.async_copy(ones, degacc.at[didx.at[j]],
                                         degsem, add=True))

            for j in range(GRP):
                p = j % NBUF
                if j >= NBUF:
                    sd[j - NBUF].wait()
                gd[j] = pltpu.async_copy(x_hbm.at[sidx.at[j]], bufs[p],
                                         gsems[p])
                if j >= DEPTH:
                    fire_scatter(j - DEPTH)
            for j in range(GRP - DEPTH, GRP):
                fire_scatter(j)
            for j in range(GRP - NBUF, GRP):
                sd[j].wait()
            for d in degd:
                d.wait()
            return carry
        lax.fori_loop(0, NGRP, group, 0)
        plsc.subcore_barrier()

        for k in range(RPT // CHUNK):
            sl = pl.ds(row0 + k * CHUNK, CHUNK)
            pltpu.sync_copy(acc.at[sl, :], sums_out.at[cid, sl, :])
        if compute_deg:
            pltpu.sync_copy(degacc.at[pl.ds(row0, RPT)],
                            deg_out.at[cid, pl.ds(row0, RPT)])

    return pl.kernel(body, out_type=tuple(out_type), mesh=mesh,
                     scratch_types=tuple(scratch))


def _tc_body(sums_ref, deg0_ref, deg1_ref, x_ref, w_ref, r_ref, b_ref, o_ref,
             *, relu):
    cnt = deg0_ref[...] + deg1_ref[...]
    inv = (1.0 / jnp.maximum(cnt, 1.0)).reshape(1, BLK)
    ii = lax.broadcasted_iota(jnp.int32, (BLK, BLK), 0)
    jj = lax.broadcasted_iota(jnp.int32, (BLK, BLK), 1)
    dm = jnp.where(ii == jj, jnp.broadcast_to(inv, (BLK, BLK)), 0.0)
    invcol = jnp.sum(dm, axis=1, keepdims=True)
    s = sums_ref[0] + sums_ref[1]
    mean = s * invcol
    acc = jnp.dot(mean, w_ref[...], preferred_element_type=jnp.float32)
    acc = acc + jnp.dot(x_ref[...], r_ref[...],
                        preferred_element_type=jnp.float32)
    acc = acc + b_ref[...]
    if relu:
        acc = jnp.maximum(acc, 0.0)
    o_ref[...] = acc


def _tc_layer(relu):
    grid = NPAD // BLK
    return pl.pallas_call(
        functools.partial(_tc_body, relu=relu),
        grid=(grid,),
        in_specs=[
            pl.BlockSpec((2, BLK, D), lambda i: (0, i, 0)),
            pl.BlockSpec((BLK,), lambda i: (i,)),
            pl.BlockSpec((BLK,), lambda i: (i,)),
            pl.BlockSpec((BLK, D), lambda i: (i, 0)),
            pl.BlockSpec((D, D), lambda i: (0, 0)),
            pl.BlockSpec((D, D), lambda i: (0, 0)),
            pl.BlockSpec((1, D), lambda i: (0, 0)),
        ],
        out_specs=pl.BlockSpec((BLK, D), lambda i: (i, 0)),
        out_shape=jax.ShapeDtypeStruct((NPAD, D), jnp.float32),
    )


def kernel(x, edge_index, edge_type, W1, root1, b1, W2, root2, b2):
    del edge_type
    src = edge_index[0]
    dst = edge_index[1]
    pad = EPAD - E
    srcp = jnp.concatenate([src, jnp.zeros((pad,), jnp.int32)])
    dstp = jnp.concatenate([dst, jnp.full((pad,), N, jnp.int32)])
    srcp = srcp.reshape(EPAD // CHUNK, CHUNK)
    dstp = dstp.reshape(EPAD // CHUNK, CHUNK)
    xpad = jnp.concatenate([x, jnp.zeros((NPAD - N, D), x.dtype)])

    res = _sc_segsum(True)(xpad, srcp, dstp)
    sums1, deg = res
    deg0 = deg[0]
    deg1 = deg[1]
    h = _tc_layer(True)(sums1, deg0, deg1, xpad, W1[0], root1,
                        b1.reshape(1, D))
    res2 = _sc_segsum(False)(h, srcp, dstp)
    sums2 = res2[0] if isinstance(res2, (tuple, list)) else res2
    out = _tc_layer(False)(sums2, deg0, deg1, h, W2[0], root2,
                           b2.reshape(1, D))
    return out[:N]


# speedup vs baseline: 13.3067x; 1.0025x over previous
import functools

import jax
import jax.numpy as jnp
from jax import lax
from jax.experimental import pallas as pl
from jax.experimental.pallas import tpu as pltpu
from jax.experimental.pallas import tpu_sc as plsc

N = 10000
D = 128
E = 320000
NPAD = 10240
EPAD = 327680
CHUNK = 64
CPW = 160
GRP = 16
NGRP = CPW // GRP
NBUF = 5
DEPTH = 3
NTILE = 16
RPT = NPAD // NTILE
BLK = 512
LANES = 16
ZR = 16


def _sc_segsum(compute_deg):
    mesh = plsc.VectorSubcoreMesh(core_axis_name="c", subcore_axis_name="s")
    out_type = [jax.ShapeDtypeStruct((2, NPAD, D), jnp.float32)]
    scratch = [
        pltpu.VMEM((GRP, CHUNK), jnp.int32),
        pltpu.VMEM((GRP, CHUNK), jnp.int32),
        pltpu.VMEM((ZR, D), jnp.float32),
        pltpu.VMEM_SHARED((NPAD, D), jnp.float32),
    ]
    scratch += [pltpu.VMEM((CHUNK, D), jnp.float32)] * NBUF
    if compute_deg:
        out_type.append(jax.ShapeDtypeStruct((2, NPAD), jnp.float32))
        scratch.append(pltpu.VMEM((CHUNK,), jnp.float32))
        scratch.append(pltpu.VMEM_SHARED((NPAD,), jnp.float32))
    scratch += [pltpu.SemaphoreType.DMA] * (2 * NBUF + 1)

    def body(x_hbm, src_hbm, dst_hbm, *rest):
        if compute_deg:
            (sums_out, deg_out, sidx, didx, zrow, acc, *tl) = rest
            bufs = tl[:NBUF]
            ones, degacc = tl[NBUF], tl[NBUF + 1]
            sems = tl[NBUF + 2:]
        else:
            (sums_out, sidx, didx, zrow, acc, *tl) = rest
            bufs = tl[:NBUF]
            deg_out = ones = degacc = None
            sems = tl[NBUF:]
        gsems = sems[:NBUF]
        ssems = sems[NBUF:2 * NBUF]
        degsem = sems[2 * NBUF]
        cid = lax.axis_index("c")
        tid = lax.axis_index("s")

        def zfill(i, carry):
            for j in range(D // LANES):
                zrow[i, pl.ds(j * LANES, LANES)] = jnp.zeros((LANES,),
                                                             jnp.float32)
            return carry
        lax.fori_loop(0, ZR, zfill, 0)
        if compute_deg:
            for j in range(CHUNK // LANES):
                ones[pl.ds(j * LANES, LANES)] = jnp.full((LANES,), 1.0,
                                                         jnp.float32)

        row0 = tid * RPT

        def zacc(k, carry):
            pltpu.sync_copy(zrow, acc.at[pl.ds(row0 + k * ZR, ZR), :])
            return carry
        lax.fori_loop(0, RPT // ZR, zacc, 0)
        if compute_deg:
            for k in range(RPT // D):
                pltpu.sync_copy(zrow.at[0],
                                degacc.at[pl.ds(row0 + k * D, D)])
        plsc.subcore_barrier()

        w = cid * NTILE + tid

        def group(g, carry):
            grow = w * CPW + g * GRP
            pltpu.sync_copy(src_hbm.at[pl.ds(grow, GRP), :], sidx)
            pltpu.sync_copy(dst_hbm.at[pl.ds(grow, GRP), :], didx)
            gd = [None] * GRP
            sd = [None] * GRP
            degd = []

            def fire_scatter(j):
                p = j % NBUF
                if gd[j] is not None:
                    gd[j].wait()
                sd[j] = pltpu.async_copy(bufs[p], acc.at[didx.at[j]],
                                         ssems[p], add=True)
                if compute_deg:
                    degd.append(
                        pltpu.async_copy(ones, degacc.at[didx.at[j]],
                                         degsem, add=True))

            for j in range(GRP):
                p = j % NBUF
                if j >= NBUF and sd[j - NBUF] is not None:
                    sd[j - NBUF].wait()
                gd[j] = pltpu.async_copy(x_hbm.at[sidx.at[j]], bufs[p],
                                         gsems[p])
                if j >= DEPTH:
                    fire_scatter(j - DEPTH)
            for j in range(GRP - DEPTH, GRP):
                fire_scatter(j)
            for j in range(GRP - NBUF, GRP):
                if sd[j] is not None:
                    sd[j].wait()
            for d in degd:
                d.wait()
            return carry
        lax.fori_loop(0, NGRP, group, 0)
        plsc.subcore_barrier()

        for k in range(RPT // CHUNK):
            sl = pl.ds(row0 + k * CHUNK, CHUNK)
            pltpu.sync_copy(acc.at[sl, :], sums_out.at[cid, sl, :])
        if compute_deg:
            pltpu.sync_copy(degacc.at[pl.ds(row0, RPT)],
                            deg_out.at[cid, pl.ds(row0, RPT)])

    return pl.kernel(body, out_type=tuple(out_type), mesh=mesh,
                     scratch_types=tuple(scratch))


def _tc_body(sums_ref, deg0_ref, deg1_ref, x_ref, w_ref, r_ref, b_ref, o_ref,
             *, relu):
    cnt = deg0_ref[...] + deg1_ref[...]
    inv = (1.0 / jnp.maximum(cnt, 1.0)).reshape(1, BLK)
    ii = lax.broadcasted_iota(jnp.int32, (BLK, BLK), 0)
    jj = lax.broadcasted_iota(jnp.int32, (BLK, BLK), 1)
    dm = jnp.where(ii == jj, jnp.broadcast_to(inv, (BLK, BLK)), 0.0)
    invcol = jnp.sum(dm, axis=1, keepdims=True)
    s = sums_ref[0] + sums_ref[1]
    mean = s * invcol
    acc = jnp.dot(mean, w_ref[...], preferred_element_type=jnp.float32)
    acc = acc + jnp.dot(x_ref[...], r_ref[...],
                        preferred_element_type=jnp.float32)
    acc = acc + b_ref[...]
    if relu:
        acc = jnp.maximum(acc, 0.0)
    o_ref[...] = acc


def _tc_layer(relu):
    grid = NPAD // BLK
    return pl.pallas_call(
        functools.partial(_tc_body, relu=relu),
        grid=(grid,),
        in_specs=[
            pl.BlockSpec((2, BLK, D), lambda i: (0, i, 0)),
            pl.BlockSpec((BLK,), lambda i: (i,)),
            pl.BlockSpec((BLK,), lambda i: (i,)),
            pl.BlockSpec((BLK, D), lambda i: (i, 0)),
            pl.BlockSpec((D, D), lambda i: (0, 0)),
            pl.BlockSpec((D, D), lambda i: (0, 0)),
            pl.BlockSpec((1, D), lambda i: (0, 0)),
        ],
        out_specs=pl.BlockSpec((BLK, D), lambda i: (i, 0)),
        out_shape=jax.ShapeDtypeStruct((NPAD, D), jnp.float32),
    )


def kernel(x, edge_index, edge_type, W1, root1, b1, W2, root2, b2):
    del edge_type
    src = edge_index[0]
    dst = edge_index[1]
    pad = EPAD - E
    srcp = jnp.concatenate([src, jnp.zeros((pad,), jnp.int32)])
    dstp = jnp.concatenate([dst, jnp.full((pad,), N, jnp.int32)])
    srcp = srcp.reshape(EPAD // CHUNK, CHUNK)
    dstp = dstp.reshape(EPAD // CHUNK, CHUNK)
    xpad = jnp.concatenate([x, jnp.zeros((NPAD - N, D), x.dtype)])

    res = _sc_segsum(True)(xpad, srcp, dstp)
    sums1, deg = res
    deg0 = deg[0]
    deg1 = deg[1]
    h = _tc_layer(True)(sums1, deg0, deg1, xpad, W1[0], root1,
                        b1.reshape(1, D))
    res2 = _sc_segsum(False)(h, srcp, dstp)
    sums2 = res2[0] if isinstance(res2, (tuple, list)) else res2
    out = _tc_layer(False)(sums2, deg0, deg1, h, W2[0], root2,
                           b2.reshape(1, D))
    return out[:N]
